# baseline (speedup 1.0000x reference)
import jax
import jax.numpy as jnp
from jax.experimental import pallas as pl
from jax.experimental.pallas import tpu as pltpu

M = 10000
B = 4096
ROW = 3072


def _body(src_ref, mem_ref, x_ref, lab_ref, y_ref, out_ref, outlab_ref):
    m = pl.program_id(0)
    s = src_ref[m]
    ovw = s < B
    out_ref[...] = jnp.where(ovw, x_ref[...], mem_ref[...])
    outlab_ref[...] = jnp.where(ovw, y_ref[...], lab_ref[...])


def kernel(mem, buffer_label, idx_keys, idx_vals, x, y):
    mem2 = mem.reshape(M, 1, ROW)
    x2 = x.reshape(B, 1, ROW)
    lab2 = buffer_label.reshape(M, 1, 1)
    y2 = y.reshape(B, 1, 1)

    wj = jnp.full((M,), -1, jnp.int32).at[idx_keys].max(
        jnp.arange(B, dtype=jnp.int32))
    src = jnp.where(wj >= 0, idx_vals[jnp.maximum(wj, 0)], B).astype(jnp.int32)

    def xmap(m, src_ref):
        return (jnp.minimum(src_ref[m], B - 1), 0, 0)

    def selfmap(m, src_ref):
        return (m, 0, 0)

    grid_spec = pltpu.PrefetchScalarGridSpec(
        num_scalar_prefetch=1,
        grid=(M,),
        in_specs=[
            pl.BlockSpec((1, 1, ROW), selfmap),
            pl.BlockSpec((1, 1, ROW), xmap),
            pl.BlockSpec((1, 1, 1), selfmap),
            pl.BlockSpec((1, 1, 1), xmap),
        ],
        out_specs=[
            pl.BlockSpec((1, 1, ROW), selfmap),
            pl.BlockSpec((1, 1, 1), selfmap),
        ],
    )

    out, outlab = pl.pallas_call(
        _body,
        grid_spec=grid_spec,
        out_shape=[
            jax.ShapeDtypeStruct((M, 1, ROW), jnp.float32),
            jax.ShapeDtypeStruct((M, 1, 1), buffer_label.dtype),
        ],
        compiler_params=pltpu.CompilerParams(
            dimension_semantics=("arbitrary",),
        ),
    )(src, mem2, x2, lab2, y2)

    return out.reshape(mem.shape), outlab.reshape(buffer_label.shape)


# speedup vs baseline: 1.3928x; 1.3928x over previous
import jax
import jax.numpy as jnp
from jax import lax
from jax.experimental import pallas as pl
from jax.experimental.pallas import tpu as pltpu
from jax.experimental.pallas import tpu_sc as plsc

M = 10000
B = 4096
ROW = 3072

NW = 32
UNITS_A = 20
UNITS_B = 19
SPLIT = 17
BASE_B = SPLIT * UNITS_A * 16
CAP = UNITS_A * 16 + 16


def _sc_body(mem_h, x_h, src_h, lab_h, y_h, out_h, olab_h,
             src_v, lrows, lsrcs, kbuf, sbuf, rowbuf,
             y_v, lab_v, olab_v, sem_c, sem_g, sem_s):
    wid = lax.axis_index("s") * 2 + lax.axis_index("c")

    def work(units, base):
        n = units * 16
        copy = pltpu.make_async_copy(
            mem_h.at[pl.ds(base, n)], out_h.at[pl.ds(base, n)], sem_c)
        copy.start()

        pltpu.sync_copy(src_h.at[pl.ds(base, n)], src_v.at[pl.ds(0, n)])
        pltpu.sync_copy(y_h, y_v)
        pltpu.sync_copy(lab_h.at[pl.ds(base, n)], lab_v.at[pl.ds(0, n)])

        lanes = lax.iota(jnp.int32, 16)
        cur = jnp.int32(0)
        for i in range(units):
            s16 = src_v[pl.ds(i * 16, 16)]
            ovw = s16 < B
            rows16 = lanes + (base + i * 16)
            csum = plsc.cumsum(ovw.astype(jnp.int32))
            pos16 = cur + csum - 1
            plsc.store_scatter(lrows, [pos16], rows16, mask=ovw)
            plsc.store_scatter(lsrcs, [pos16], s16, mask=ovw)
            cur = cur + csum[15]
            g = plsc.load_gather(y_v, [jnp.minimum(s16, B - 1)])
            olab_v[pl.ds(i * 16, 16)] = jnp.where(ovw, g, lab_v[pl.ds(i * 16, 16)])
        pltpu.sync_copy(olab_v.at[pl.ds(0, n)], olab_h.at[pl.ds(base, n)])

        @pl.when(cur > 0)
        def _pad():
            zero16 = jnp.zeros((16,), jnp.int32)
            r0 = plsc.load_gather(lrows, [zero16])
            s0 = plsc.load_gather(lsrcs, [zero16])
            lrows[pl.ds(cur, 16)] = r0
            lsrcs[pl.ds(cur, 16)] = s0

        copy.wait()
        nc = (cur + 15) // 16

        def chunk(c, carry):
            kbuf[...] = lrows[pl.ds(c * 16, 16)]
            sbuf[...] = lsrcs[pl.ds(c * 16, 16)]
            pltpu.async_copy(x_h.at[sbuf], rowbuf, sem_g).wait()
            pltpu.async_copy(rowbuf, out_h.at[kbuf], sem_s).wait()
            return carry

        lax.fori_loop(0, nc, chunk, jnp.int32(0))

    @pl.when(wid < SPLIT)
    def _a():
        work(UNITS_A, wid * (UNITS_A * 16))

    @pl.when(wid >= SPLIT)
    def _b():
        work(UNITS_B, BASE_B + (wid - SPLIT) * (UNITS_B * 16))


def kernel(mem, buffer_label, idx_keys, idx_vals, x, y):
    mem2 = mem.reshape(M, ROW)
    x2 = x.reshape(B, ROW)

    wj = jnp.full((M,), -1, jnp.int32).at[idx_keys].max(
        jnp.arange(B, dtype=jnp.int32))
    src = jnp.where(wj >= 0, idx_vals[jnp.maximum(wj, 0)], B).astype(jnp.int32)

    mesh = plsc.VectorSubcoreMesh(core_axis_name="c", subcore_axis_name="s")
    out, olab = pl.kernel(
        _sc_body,
        mesh=mesh,
        compiler_params=pltpu.CompilerParams(needs_layout_passes=False),
        out_type=[
            jax.ShapeDtypeStruct((M, ROW), jnp.float32),
            jax.ShapeDtypeStruct((M,), buffer_label.dtype),
        ],
        scratch_types=[
            pltpu.VMEM((UNITS_A * 16,), jnp.int32),
            pltpu.VMEM((CAP,), jnp.int32),
            pltpu.VMEM((CAP,), jnp.int32),
            pltpu.VMEM((16,), jnp.int32),
            pltpu.VMEM((16,), jnp.int32),
            pltpu.VMEM((16, ROW), jnp.float32),
            pltpu.VMEM((B,), jnp.int32),
            pltpu.VMEM((UNITS_A * 16,), jnp.int32),
            pltpu.VMEM((UNITS_A * 16,), jnp.int32),
            pltpu.SemaphoreType.DMA,
            pltpu.SemaphoreType.DMA,
            pltpu.SemaphoreType.DMA,
        ],
    )(mem2, x2, src, buffer_label, y)

    return out.reshape(mem.shape), olab.reshape(buffer_label.shape)
